# Initial kernel scaffold; baseline (speedup 1.0000x reference)
import jax
import jax.numpy as jnp
from jax.experimental import pallas as pl


def kernel(logits, targets):
    raise NotImplementedError("write your pallas kernel here")


# speedup vs baseline: 155.7145x; 155.7145x over previous
import functools
import math

import jax
import jax.numpy as jnp
from jax.experimental import pallas as pl
from jax.experimental.pallas import tpu as pltpu

N_TOK = 4096
VOCAB = 32000
BT = 512
VC = 3200
T_BLOCKS = N_TOK // BT
V_CHUNKS = VOCAB // VC

MIN_TOKENS_RATIO = 0.25
WARMUP_STEPS = 1000
THRESHOLD_SENSITIVITY = 0.5
STEP_COUNT = 0

_NEG_INF = float("-inf")


def _stats_kernel(x_ref, tgt_ref, nll_ref, conf_ref, diff_ref,
                  m1_ref, m2_ref, s_ref, t_ref, tg_ref):
    j = pl.program_id(1)

    @pl.when(j == 0)
    def _init():
        m1_ref[...] = jnp.full((BT, 1), _NEG_INF, jnp.float32)
        m2_ref[...] = jnp.full((BT, 1), _NEG_INF, jnp.float32)
        s_ref[...] = jnp.zeros((BT, 1), jnp.float32)
        t_ref[...] = jnp.zeros((BT, 1), jnp.float32)
        tg_ref[...] = jnp.zeros((BT, 1), jnp.float32)

    x = x_ref[...]

    cm1 = jnp.max(x, axis=1, keepdims=True)
    is_max = x == cm1
    n_max = jnp.sum(is_max.astype(jnp.float32), axis=1, keepdims=True)
    cm2 = jnp.max(jnp.where(is_max, _NEG_INF, x), axis=1, keepdims=True)
    cm2 = jnp.where(n_max > 1.0, cm1, cm2)

    m1 = m1_ref[...]
    m2 = m2_ref[...]
    new_m1 = jnp.maximum(m1, cm1)
    new_m2 = jnp.maximum(jnp.minimum(m1, cm1), jnp.maximum(m2, cm2))

    e = jnp.exp(x - new_m1)
    cs = jnp.sum(e, axis=1, keepdims=True)
    ct = jnp.sum(e * x, axis=1, keepdims=True)
    alpha = jnp.where(m1 == _NEG_INF, 0.0, jnp.exp(m1 - new_m1))
    s = s_ref[...] * alpha + cs
    t = t_ref[...] * alpha + ct

    tgt = tgt_ref[0, 0, :]
    cols = jax.lax.broadcasted_iota(jnp.int32, (BT, VC), 1) + j * VC
    hit = cols == tgt[:, None]
    tg = tg_ref[...] + jnp.sum(jnp.where(hit, x, 0.0), axis=1, keepdims=True)

    m1_ref[...] = new_m1
    m2_ref[...] = new_m2
    s_ref[...] = s
    t_ref[...] = t
    tg_ref[...] = tg

    @pl.when(j == V_CHUNKS - 1)
    def _finalize():
        lse = new_m1 + jnp.log(s)
        nll = lse - tg
        entropy = lse - t / s
        conf = jnp.exp(new_m1 - lse)
        p2 = jnp.exp(new_m2 - lse)
        margin = conf - p2
        difficulty = entropy / math.log(float(VOCAB)) + (1.0 - margin) + nll
        nll_ref[0, 0, :] = nll[:, 0]
        conf_ref[0, 0, :] = conf[:, 0]
        diff_ref[0, 0, :] = difficulty[:, 0]


def _mask_loss_kernel(d_row_ref, d_col_ref, nll_col_ref, conf_ref, out_ref):
    conf = conf_ref[...]
    avg_conf = jnp.sum(conf) / float(N_TOK)
    progress = min(1.0, STEP_COUNT / max(1, WARMUP_STEPS))
    base_ratio = 1.0 - progress * (1.0 - MIN_TOKENS_RATIO)
    ratio = jnp.clip(
        base_ratio * (1.0 + THRESHOLD_SENSITIVITY * (1.0 - 2.0 * avg_conf)),
        0.05, 1.0)
    k = jnp.maximum(1, jnp.floor(ratio * float(N_TOK)).astype(jnp.int32))
    kf = k.astype(jnp.float32)

    d_row = d_row_ref[...]
    RB = 512
    total = jnp.zeros((1, 1), jnp.float32)
    for b in range(N_TOK // RB):
        db = d_col_ref[pl.ds(b * RB, RB), :]
        nb = nll_col_ref[pl.ds(b * RB, RB), :]
        gt = (d_row > db).astype(jnp.float32)
        jj = jax.lax.broadcasted_iota(jnp.int32, (RB, N_TOK), 1)
        ii = jax.lax.broadcasted_iota(jnp.int32, (RB, N_TOK), 0) + b * RB
        tie = jnp.logical_and(d_row == db, jj < ii).astype(jnp.float32)
        rank = jnp.sum(gt + tie, axis=1, keepdims=True)
        sel = (rank < kf).astype(jnp.float32)
        total = total + jnp.sum(nb * sel, axis=0, keepdims=True)
    out_ref[...] = total / jnp.maximum(kf, 1.0)


@functools.partial(jax.jit, static_argnames=())
def kernel(logits, targets):
    lf = logits.reshape(N_TOK, VOCAB)
    tgt = targets.reshape(T_BLOCKS, 1, BT).astype(jnp.int32)

    nll, conf, diff = pl.pallas_call(
        _stats_kernel,
        grid=(T_BLOCKS, V_CHUNKS),
        in_specs=[
            pl.BlockSpec((BT, VC), lambda i, j: (i, j)),
            pl.BlockSpec((1, 1, BT), lambda i, j: (i, 0, 0)),
        ],
        out_specs=[
            pl.BlockSpec((1, 1, BT), lambda i, j: (i, 0, 0)),
            pl.BlockSpec((1, 1, BT), lambda i, j: (i, 0, 0)),
            pl.BlockSpec((1, 1, BT), lambda i, j: (i, 0, 0)),
        ],
        out_shape=[
            jax.ShapeDtypeStruct((T_BLOCKS, 1, BT), jnp.float32),
            jax.ShapeDtypeStruct((T_BLOCKS, 1, BT), jnp.float32),
            jax.ShapeDtypeStruct((T_BLOCKS, 1, BT), jnp.float32),
        ],
        scratch_shapes=[pltpu.VMEM((BT, 1), jnp.float32) for _ in range(5)],
    )(lf, tgt)

    d_row = diff.reshape(1, N_TOK)
    d_col = diff.reshape(N_TOK, 1)
    nll_col = nll.reshape(N_TOK, 1)
    conf_row = conf.reshape(1, N_TOK)

    loss = pl.pallas_call(
        _mask_loss_kernel,
        out_shape=jax.ShapeDtypeStruct((1, 1), jnp.float32),
    )(d_row, d_col, nll_col, conf_row)
    return loss.reshape(())
